# baseline (speedup 1.0000x reference)
import dataclasses
import functools

import jax
import jax.numpy as jnp
import numpy as np
from jax import lax
from jax.experimental import pallas as pl
from jax.experimental.pallas import tpu as pltpu
from jax.experimental.pallas import tpu_sc as plsc

N = 10000
E = 320000
DIN = 128
H = 128

NTILES = 32
NPT = 313
NROWS = NPT + 1
MAGIC = 13401
MSHIFT = 22
PACK_SHIFT = 14
PAD_PACKED = NPT << PACK_SHIFT

CH = 2000
NCH = E // CH
BR = 128
BATCH_PAIR = 2 * BR
FLUSH = 2048
BUFCAP = 4096

NOUT_PAD = NTILES * NPT + 1
ACC = NROWS * 128

_BLK = 1000

_mesh = plsc.VectorSubcoreMesh(core_axis_name="c", subcore_axis_name="s")


def _sc_params():
    cp = pltpu.CompilerParams()
    if "needs_layout_passes" in pltpu.CompilerParams.__dataclass_fields__:
        cp = dataclasses.replace(cp, needs_layout_passes=False)
    return cp


def _wid():
    return lax.axis_index("s") * 2 + lax.axis_index("c")


def _group_body(src_hbm, dst_hbm, pe_hbm, cnts_hbm, srcbuf, dstbuf,
                srcbuf2, dstbuf2, buf, outv, gsem0, gsem1):
    w = _wid()
    w_vec = jnp.full((16,), 0, jnp.int32) + w
    base_vec = w_vec * NPT

    def start_load(c, sb, db, sem):
        off = c * CH
        pltpu.async_copy(src_hbm.at[pl.ds(off, CH)], sb, sem)
        pltpu.async_copy(dst_hbm.at[pl.ds(off, CH)], db, sem)

    def wait_load(sb, db, sem):
        pltpu.make_async_copy(src_hbm.at[pl.ds(0, CH)], sb, sem).wait()
        pltpu.make_async_copy(dst_hbm.at[pl.ds(0, CH)], db, sem).wait()

    def chunk_body(c, carry, sb, db):
        cursor_vec, hbm_cur = carry

        def vreg_body(j, cur):
            s16 = sb[pl.ds(j * 16, 16)]
            d16 = db[pl.ds(j * 16, 16)]
            own = jnp.right_shift(d16 * MAGIC, MSHIFT)
            m = own == w_vec
            dl = d16 - base_vec
            pk = jnp.left_shift(dl, PACK_SHIFT) | s16
            cum = plsc.cumsum(m.astype(jnp.int32))
            pos = cur + cum - 1
            plsc.store_scatter(buf, [pos], pk, mask=m)
            return cur + plsc.all_reduce_population_count(m)

        cursor_vec = lax.fori_loop(0, CH // 16, vreg_body, cursor_vec, unroll=8)
        cursor_s = jnp.max(cursor_vec, axis=0)
        w_unused = 0
        do_flush = cursor_s >= FLUSH

        @pl.when(do_flush)
        def _():
            pltpu.sync_copy(buf.at[pl.ds(0, FLUSH)],
                            pe_hbm.at[pl.ds(pl.multiple_of(w * E + hbm_cur, FLUSH), FLUSH)])
            nshift = (cursor_s - FLUSH + 15) >> 4

            def shift_body(j, _):
                v = buf[pl.ds(FLUSH + j * 16, 16)]
                buf[pl.ds(j * 16, 16)] = v
                return 0

            lax.fori_loop(0, nshift, shift_body, 0)

        dec = jnp.where(do_flush, FLUSH, 0)
        return cursor_vec - dec, hbm_cur + dec

    def pair_body(i, carry):
        start_load(2 * i + 1, srcbuf2, dstbuf2, gsem1)
        wait_load(srcbuf, dstbuf, gsem0)
        carry = chunk_body(2 * i, carry, srcbuf, dstbuf)

        @pl.when(i < NCH // 2 - 1)
        def _():
            start_load(2 * i + 2, srcbuf, dstbuf, gsem0)

        wait_load(srcbuf2, dstbuf2, gsem1)
        return chunk_body(2 * i + 1, carry, srcbuf2, dstbuf2)

    start_load(0, srcbuf, dstbuf, gsem0)
    cursor_vec, hbm_cur = lax.fori_loop(
        0, NCH // 2, pair_body,
        (jnp.zeros((16,), jnp.int32), jnp.int32(0)))

    cursor_s = jnp.max(cursor_vec, axis=0)
    total = hbm_cur + cursor_s
    target = (total + BATCH_PAIR - 1) & ~(BATCH_PAIR - 1)
    cursor_end = target - hbm_cur
    astart = (cursor_s >> 4) << 4
    npadv = (cursor_end - astart + 15) >> 4
    iota16 = lax.iota(jnp.int32, 16)

    def pad_body(j, _):
        a = astart + j * 16
        v = buf[pl.ds(a, 16)]
        keep = (a + iota16) < cursor_s
        buf[pl.ds(a, 16)] = jnp.where(keep, v, jnp.full((16,), PAD_PACKED,
                                                        jnp.int32))
        return 0

    lax.fori_loop(0, npadv, pad_body, 0)

    nf = cursor_end >> 8

    def flush_body(j, _):
        pltpu.sync_copy(buf.at[pl.ds(pl.multiple_of(j * 256, 256), 256)],
                        pe_hbm.at[pl.ds(pl.multiple_of(w * E + hbm_cur + j * 256, 256), 256)])
        return 0

    lax.fori_loop(0, nf, flush_body, 0)

    outv[...] = jnp.zeros((16,), jnp.int32) + target
    pltpu.sync_copy(outv, cnts_hbm.at[pl.ds(pl.multiple_of(w * 16, 16), 16)])


def _sc_group(src, dst):
    k = pl.kernel(
        _group_body,
        out_type=[
            jax.ShapeDtypeStruct((NTILES * E,), jnp.int32),
            jax.ShapeDtypeStruct((NTILES * 16,), jnp.int32),
        ],
        mesh=_mesh,
        scratch_types=[
            pltpu.VMEM((CH,), jnp.int32),
            pltpu.VMEM((CH,), jnp.int32),
            pltpu.VMEM((CH,), jnp.int32),
            pltpu.VMEM((CH,), jnp.int32),
            pltpu.VMEM((BUFCAP,), jnp.int32),
            pltpu.VMEM((16,), jnp.int32),
            pltpu.SemaphoreType.DMA,
            pltpu.SemaphoreType.DMA,
        ],
        compiler_params=_sc_params(),
    )
    return k(src, dst)


def _make_agg_body(with_counts):
    SUP = 2048

    def body(h_hbm, pe_hbm, cnts_hbm, *refs):
        if with_counts:
            (mmax_hbm, msum_hbm, cnt_hbm, accM, accS, accC,
             pkS, idxS, rows0, rows1, cbuf, sem0, sem1) = refs
        else:
            (mmax_hbm, msum_hbm, accM, accS,
             pkS, idxS, rows0, rows1, cbuf, sem0, sem1) = refs
        w = _wid()
        neg_inf = jnp.full((16,), -jnp.inf, jnp.float32)
        zeros = jnp.zeros((16,), jnp.float32)
        ones = jnp.ones((16,), jnp.float32)
        fmask = jnp.full((16,), (1 << PACK_SHIFT) - 1, jnp.int32)

        @pl.loop(0, ACC, step=16)
        def _(i):
            accM[pl.ds(i, 16)] = neg_inf
            accS[pl.ds(i, 16)] = zeros

        if with_counts:
            @pl.loop(0, NROWS * 16, step=16)
            def _(i):
                accC[pl.ds(i, 16)] = zeros

        pltpu.sync_copy(cnts_hbm.at[pl.ds(pl.multiple_of(w * 16, 16), 16)],
                        cbuf)
        npairs = jnp.max(cbuf[...], axis=0) >> 8
        nsup = npairs >> 3
        rem = npairs & 7

        def unpack(n):
            @pl.loop(0, n, step=16)
            def _(j):
                idxS[pl.ds(j, 16)] = pkS[pl.ds(j, 16)] & fmask

        def gather_start(b, rows, sem):
            pltpu.async_copy(h_hbm.at[idxS.at[pl.ds(b * BR, BR)]], rows, sem)

        def gather_wait(rows, sem):
            pltpu.make_async_copy(h_hbm.at[idxS.at[pl.ds(0, BR)]], rows,
                                  sem).wait()

        def process(rows, pkbase):
            def _pgroup(g, _):
                pk16 = pkS[pl.ds(pkbase + g * 16, 16)]
                for l in range(16):
                    p = pk16[l]
                    e = g * 16 + l
                    fb = jnp.right_shift(p, PACK_SHIFT) << 7
                    for kk in range(8):
                        sl = pl.ds(fb + kk * 16, 16)
                        r = rows[e, pl.ds(kk * 16, 16)]
                        accM[sl] = jnp.maximum(accM[sl], r)
                        plsc.addupdate(accS.at[sl], r)
                    if with_counts:
                        cs = pl.ds(jnp.right_shift(p, PACK_SHIFT) << 4, 16)
                        plsc.addupdate(accC.at[cs], ones)
                return 0

            lax.fori_loop(0, BR // 16, _pgroup, 0, unroll=2)

        def super_body(s, _):
            pltpu.sync_copy(
                pe_hbm.at[pl.ds(pl.multiple_of(w * E + s * SUP, SUP), SUP)],
                pkS)
            unpack(SUP)
            gather_start(0, rows0, sem0)

            def bpair(bp, _):
                gather_wait(rows0, sem0)
                gather_start(2 * bp + 1, rows1, sem1)
                process(rows0, (2 * bp) * BR)
                gather_wait(rows1, sem1)

                @pl.when(bp < SUP // (2 * BR) - 1)
                def _():
                    gather_start(2 * bp + 2, rows0, sem0)

                process(rows1, (2 * bp + 1) * BR)
                return 0

            lax.fori_loop(0, SUP // (2 * BR), bpair, 0)
            return 0

        lax.fori_loop(0, nsup, super_body, 0)

        def tail_body(t, _):
            base_e = nsup * SUP + t * (2 * BR)
            pltpu.sync_copy(
                pe_hbm.at[pl.ds(pl.multiple_of(w * E + base_e, 2 * BR),
                                2 * BR)],
                pkS.at[pl.ds(0, 2 * BR)])
            unpack(2 * BR)
            gather_start(0, rows0, sem0)
            gather_wait(rows0, sem0)
            gather_start(1, rows1, sem1)
            process(rows0, 0)
            gather_wait(rows1, sem1)
            process(rows1, BR)
            return 0

        lax.fori_loop(0, rem, tail_body, 0)

        woff = w * (NPT * 128)
        pltpu.sync_copy(accM.at[pl.ds(0, NPT * 128)],
                        mmax_hbm.at[pl.ds(pl.multiple_of(woff, 128),
                                          NPT * 128)])
        pltpu.sync_copy(accS.at[pl.ds(0, NPT * 128)],
                        msum_hbm.at[pl.ds(pl.multiple_of(woff, 128),
                                          NPT * 128)])
        if with_counts:
            pltpu.sync_copy(
                accC.at[pl.ds(0, NPT * 16)],
                cnt_hbm.at[pl.ds(pl.multiple_of(w * (NPT * 16), 16),
                                 NPT * 16)])

    return body


def _sc_agg(h, pe, cnts, with_counts):
    out_type = [
        jax.ShapeDtypeStruct((NOUT_PAD * 128,), jnp.float32),
        jax.ShapeDtypeStruct((NOUT_PAD * 128,), jnp.float32),
    ]
    scratch = [
        pltpu.VMEM((ACC,), jnp.float32),
        pltpu.VMEM((ACC,), jnp.float32),
    ]
    if with_counts:
        out_type.append(jax.ShapeDtypeStruct((NOUT_PAD * 16,), jnp.float32))
        scratch.append(pltpu.VMEM((NROWS * 16,), jnp.float32))
    scratch += [
        pltpu.VMEM((2048,), jnp.int32),
        pltpu.VMEM((2048,), jnp.int32),
        pltpu.VMEM((BR, 128), jnp.float32),
        pltpu.VMEM((BR, 128), jnp.float32),
        pltpu.VMEM((16,), jnp.int32),
        pltpu.SemaphoreType.DMA,
        pltpu.SemaphoreType.DMA,
    ]
    k = pl.kernel(
        _make_agg_body(with_counts),
        out_type=out_type,
        mesh=_mesh,
        scratch_types=scratch,
        compiler_params=_sc_params(),
    )
    return k(h, pe, cnts)


def _ln(y, g, b):
    mu = jnp.mean(y, axis=-1, keepdims=True)
    d = y - mu
    var = jnp.mean(d * d, axis=-1, keepdims=True)
    return g * d / jnp.sqrt(var + 1e-5) + b


def _silu(y):
    return y * jax.nn.sigmoid(y)


_HI = jax.lax.Precision.HIGHEST


def _adapter_body(x_ref, w_ref, b_ref, o_ref):
    o_ref[...] = jnp.dot(x_ref[...], w_ref[...], precision=_HI) + b_ref[...]


def _tc_adapter(x, WaT, ba):
    return pl.pallas_call(
        _adapter_body,
        grid=(N // _BLK,),
        in_specs=[
            pl.BlockSpec((_BLK, DIN), lambda i: (i, 0)),
            pl.BlockSpec((DIN, H), lambda i: (0, 0)),
            pl.BlockSpec((1, H), lambda i: (0, 0)),
        ],
        out_specs=pl.BlockSpec((_BLK, H), lambda i: (i, 0)),
        out_shape=jax.ShapeDtypeStruct((N, H), jnp.float32),
    )(x, WaT, ba.reshape(1, H))


def _dense_body(mmax_ref, msum_ref, cnt_ref, hin_ref, res_ref, wlmax_ref,
                wlmean_ref, bl_ref, wr_ref, g_ref, b_ref, o_ref):
    mmax = mmax_ref[...]
    mmaxf = jnp.where(jnp.isneginf(mmax), 0.0, mmax)
    cnt = cnt_ref[...][:, 0:1]
    mean = msum_ref[...] / jnp.maximum(cnt, 1.0)
    y = (jnp.dot(mmaxf, wlmax_ref[...], precision=_HI)
         + jnp.dot(mean, wlmean_ref[...], precision=_HI)
         + jnp.dot(hin_ref[...], wr_ref[...], precision=_HI)
         + bl_ref[...])
    y = _ln(y, g_ref[...], b_ref[...])
    o_ref[...] = _silu(y + res_ref[...])


def _tc_dense(mmax, msum, cntc, hin, res, WlTmax, WlTmean, bl, WrT, g, b):
    return pl.pallas_call(
        _dense_body,
        grid=(N // _BLK,),
        in_specs=[
            pl.BlockSpec((_BLK, H), lambda i: (i, 0)),
            pl.BlockSpec((_BLK, H), lambda i: (i, 0)),
            pl.BlockSpec((_BLK, 16), lambda i: (i, 0)),
            pl.BlockSpec((_BLK, H), lambda i: (i, 0)),
            pl.BlockSpec((_BLK, H), lambda i: (i, 0)),
            pl.BlockSpec((H, H), lambda i: (0, 0)),
            pl.BlockSpec((H, H), lambda i: (0, 0)),
            pl.BlockSpec((1, H), lambda i: (0, 0)),
            pl.BlockSpec((H, H), lambda i: (0, 0)),
            pl.BlockSpec((1, H), lambda i: (0, 0)),
            pl.BlockSpec((1, H), lambda i: (0, 0)),
        ],
        out_specs=pl.BlockSpec((_BLK, H), lambda i: (i, 0)),
        out_shape=jax.ShapeDtypeStruct((N, H), jnp.float32),
    )(mmax, msum, cntc, hin, res, WlTmax, WlTmean, bl.reshape(1, H), WrT,
      g.reshape(1, H), b.reshape(1, H))


def _mlp_body(h_ref, w1_ref, b1_ref, g_ref, b_ref, w2_ref, b2_ref, o_ref):
    z = jnp.dot(h_ref[...], w1_ref[...], precision=_HI) + b1_ref[...]
    z = _silu(z)
    z = _ln(z, g_ref[...], b_ref[...])
    o_ref[...] = jnp.dot(z, w2_ref[...], precision=_HI) + b2_ref[...]


def _tc_mlp(h, W1T, b1m, gm, bm, W2T, b2m):
    return pl.pallas_call(
        _mlp_body,
        grid=(N // _BLK,),
        in_specs=[
            pl.BlockSpec((_BLK, H), lambda i: (i, 0)),
            pl.BlockSpec((H, 4 * H), lambda i: (0, 0)),
            pl.BlockSpec((1, 4 * H), lambda i: (0, 0)),
            pl.BlockSpec((1, 4 * H), lambda i: (0, 0)),
            pl.BlockSpec((1, 4 * H), lambda i: (0, 0)),
            pl.BlockSpec((4 * H, H), lambda i: (0, 0)),
            pl.BlockSpec((1, H), lambda i: (0, 0)),
        ],
        out_specs=pl.BlockSpec((_BLK, H), lambda i: (i, 0)),
        out_shape=jax.ShapeDtypeStruct((N, H), jnp.float32),
    )(h, W1T, b1m.reshape(1, 4 * H), gm.reshape(1, 4 * H),
      bm.reshape(1, 4 * H), W2T, b2m.reshape(1, H))


_PERM = np.arange(128).reshape(4, 2, 16).transpose(0, 2, 1).reshape(128)


def kernel(x, edge_index, Wa, ba, Wl0, bl0, Wr0, g0, b0, Wl1, bl1, Wr1, g1, b1,
           Wl2, bl2, Wr2, g2, b2, W1, b1m, gm, bm, W2, b2m):
    pe, cnts = _sc_group(edge_index[0], edge_index[1])
    x_res = _tc_adapter(x, Wa.T, ba)

    mmax_f, msum_f, cnt_f = _sc_agg(x, pe, cnts, with_counts=True)
    mmax = mmax_f[:N * 128].reshape(N, 128)
    msum = msum_f[:N * 128].reshape(N, 128)
    cntc = cnt_f[:N * 16].reshape(N, 16)

    h = _tc_dense(mmax, msum, cntc, x, x_res,
                  Wl0[:, :DIN].T, Wl0[:, DIN:].T, bl0, Wr0.T, g0, b0)
    for (Wl, bl, Wr, g, b) in ((Wl1, bl1, Wr1, g1, b1),
                               (Wl2, bl2, Wr2, g2, b2)):
        mmax_f, msum_f = _sc_agg(h, pe, cnts, with_counts=False)
        mmax = mmax_f[:N * 128].reshape(N, 128)
        msum = msum_f[:N * 128].reshape(N, 128)
        h = _tc_dense(mmax, msum, cntc, h, h,
                      Wl[:, :H].T, Wl[:, H:].T, bl, Wr.T, g, b)
    return _tc_mlp(h, W1.T, b1m, gm, bm, W2.T, b2m)


# speedup vs baseline: 1.0153x; 1.0153x over previous
import dataclasses

import jax
import jax.numpy as jnp
import numpy as np
from jax import lax
from jax.experimental import pallas as pl
from jax.experimental.pallas import tpu as pltpu
from jax.experimental.pallas import tpu_sc as plsc

N = 10000
E = 320000
DIN = 128
H = 128

NTILES = 32
NPT = 313
NROWS = NPT + 1
MAGIC = 13401
MSHIFT = 22
PACK_SHIFT = 14
PAD_PACKED = NPT << PACK_SHIFT

CH = 2000
NCH = E // CH
BR = 128
BATCH_PAIR = 2 * BR
FLUSH = 2048
BUFCAP = 4096

NOUT_PAD = NTILES * NPT + 1
ACC = NROWS * 128

_BLK = 1000

_mesh = plsc.VectorSubcoreMesh(core_axis_name="c", subcore_axis_name="s")


def _sc_params():
    cp = pltpu.CompilerParams()
    if "needs_layout_passes" in pltpu.CompilerParams.__dataclass_fields__:
        cp = dataclasses.replace(cp, needs_layout_passes=False)
    return cp


def _wid():
    return lax.axis_index("s") * 2 + lax.axis_index("c")


def _group_body(src_hbm, dst_hbm, pe_hbm, cnts_hbm, srcbuf, dstbuf,
                srcbuf2, dstbuf2, buf, outv, gsem0, gsem1):
    w = _wid()
    w_vec = jnp.full((16,), 0, jnp.int32) + w
    base_vec = w_vec * NPT

    def start_load(c, sb, db, sem):
        off = c * CH
        pltpu.async_copy(src_hbm.at[pl.ds(off, CH)], sb, sem)
        pltpu.async_copy(dst_hbm.at[pl.ds(off, CH)], db, sem)

    def wait_load(sb, db, sem):
        pltpu.make_async_copy(src_hbm.at[pl.ds(0, CH)], sb, sem).wait()
        pltpu.make_async_copy(dst_hbm.at[pl.ds(0, CH)], db, sem).wait()

    def chunk_body(c, carry, sb, db):
        cursor_vec, hbm_cur = carry

        def vreg_body(j, cur):
            s16 = sb[pl.ds(j * 16, 16)]
            d16 = db[pl.ds(j * 16, 16)]
            own = jnp.right_shift(d16 * MAGIC, MSHIFT)
            m = own == w_vec
            dl = d16 - base_vec
            pk = jnp.left_shift(dl, PACK_SHIFT) | s16
            cum = plsc.cumsum(m.astype(jnp.int32))
            pos = cur + cum - 1
            plsc.store_scatter(buf, [pos], pk, mask=m)
            return cur + plsc.all_reduce_population_count(m)

        cursor_vec = lax.fori_loop(0, CH // 16, vreg_body, cursor_vec, unroll=8)
        cursor_s = jnp.max(cursor_vec, axis=0)
        do_flush = cursor_s >= FLUSH

        @pl.when(do_flush)
        def _():
            pltpu.sync_copy(buf.at[pl.ds(0, FLUSH)],
                            pe_hbm.at[pl.ds(pl.multiple_of(w * E + hbm_cur, FLUSH), FLUSH)])
            nshift = (cursor_s - FLUSH + 15) >> 4

            def shift_body(j, _):
                v = buf[pl.ds(FLUSH + j * 16, 16)]
                buf[pl.ds(j * 16, 16)] = v
                return 0

            lax.fori_loop(0, nshift, shift_body, 0)

        dec = jnp.where(do_flush, FLUSH, 0)
        return cursor_vec - dec, hbm_cur + dec

    def pair_body(i, carry):
        start_load(2 * i + 1, srcbuf2, dstbuf2, gsem1)
        wait_load(srcbuf, dstbuf, gsem0)
        carry = chunk_body(2 * i, carry, srcbuf, dstbuf)

        @pl.when(i < NCH // 2 - 1)
        def _():
            start_load(2 * i + 2, srcbuf, dstbuf, gsem0)

        wait_load(srcbuf2, dstbuf2, gsem1)
        return chunk_body(2 * i + 1, carry, srcbuf2, dstbuf2)

    start_load(0, srcbuf, dstbuf, gsem0)
    cursor_vec, hbm_cur = lax.fori_loop(
        0, NCH // 2, pair_body,
        (jnp.zeros((16,), jnp.int32), jnp.int32(0)))

    cursor_s = jnp.max(cursor_vec, axis=0)
    total = hbm_cur + cursor_s
    target = (total + BATCH_PAIR - 1) & ~(BATCH_PAIR - 1)
    cursor_end = target - hbm_cur
    astart = (cursor_s >> 4) << 4
    npadv = (cursor_end - astart + 15) >> 4
    iota16 = lax.iota(jnp.int32, 16)

    def pad_body(j, _):
        a = astart + j * 16
        v = buf[pl.ds(a, 16)]
        keep = (a + iota16) < cursor_s
        buf[pl.ds(a, 16)] = jnp.where(keep, v, jnp.full((16,), PAD_PACKED,
                                                        jnp.int32))
        return 0

    lax.fori_loop(0, npadv, pad_body, 0)

    nf = cursor_end >> 8

    def flush_body(j, _):
        pltpu.sync_copy(buf.at[pl.ds(pl.multiple_of(j * 256, 256), 256)],
                        pe_hbm.at[pl.ds(pl.multiple_of(w * E + hbm_cur + j * 256, 256), 256)])
        return 0

    lax.fori_loop(0, nf, flush_body, 0)

    outv[...] = jnp.zeros((16,), jnp.int32) + target
    pltpu.sync_copy(outv, cnts_hbm.at[pl.ds(pl.multiple_of(w * 16, 16), 16)])


def _sc_group(src, dst):
    k = pl.kernel(
        _group_body,
        out_type=[
            jax.ShapeDtypeStruct((NTILES * E,), jnp.int32),
            jax.ShapeDtypeStruct((NTILES * 16,), jnp.int32),
        ],
        mesh=_mesh,
        scratch_types=[
            pltpu.VMEM((CH,), jnp.int32),
            pltpu.VMEM((CH,), jnp.int32),
            pltpu.VMEM((CH,), jnp.int32),
            pltpu.VMEM((CH,), jnp.int32),
            pltpu.VMEM((BUFCAP,), jnp.int32),
            pltpu.VMEM((16,), jnp.int32),
            pltpu.SemaphoreType.DMA,
            pltpu.SemaphoreType.DMA,
        ],
        compiler_params=_sc_params(),
    )
    return k(src, dst)


def _make_agg_body(with_counts):
    SUP = 2048

    def body(h_hbm, pe_hbm, cnts_hbm, *refs):
        if with_counts:
            (mmax_hbm, msum_hbm, cnt_hbm, accM, accS, accC,
             pkS, idxS, rows0, rows1, cbuf, sem0, sem1) = refs
        else:
            (mmax_hbm, msum_hbm, accM, accS,
             pkS, idxS, rows0, rows1, cbuf, sem0, sem1) = refs
        w = _wid()
        neg_inf = jnp.full((16,), -jnp.inf, jnp.float32)
        zeros = jnp.zeros((16,), jnp.float32)
        ones = jnp.ones((16,), jnp.float32)
        fmask = jnp.full((16,), (1 << PACK_SHIFT) - 1, jnp.int32)

        @pl.loop(0, ACC, step=16)
        def _(i):
            accM[pl.ds(i, 16)] = neg_inf
            accS[pl.ds(i, 16)] = zeros

        if with_counts:
            @pl.loop(0, NROWS * 16, step=16)
            def _(i):
                accC[pl.ds(i, 16)] = zeros

        pltpu.sync_copy(cnts_hbm.at[pl.ds(pl.multiple_of(w * 16, 16), 16)],
                        cbuf)
        npairs = jnp.max(cbuf[...], axis=0) >> 8
        nsup = npairs >> 3
        rem = npairs & 7

        def unpack(n):
            @pl.loop(0, n, step=16)
            def _(j):
                idxS[pl.ds(j, 16)] = pkS[pl.ds(j, 16)] & fmask

        def gather_start(b, rows, sem):
            pltpu.async_copy(h_hbm.at[idxS.at[pl.ds(b * BR, BR)]], rows, sem)

        def gather_wait(rows, sem):
            pltpu.make_async_copy(h_hbm.at[idxS.at[pl.ds(0, BR)]], rows,
                                  sem).wait()

        def process(rows, pkbase, absbase):
            def _pgroup(g, _):
                pk16 = pkS[pl.ds(pkbase + g * 16, 16)]
                for l in range(16):
                    p = pk16[l]
                    e = g * 16 + l
                    fb = jnp.right_shift(p, PACK_SHIFT) << 7
                    for kk in range(8):
                        sl = pl.ds(fb + kk * 16, 16)
                        r = rows[e, pl.ds(kk * 16, 16)]
                        accM[sl] = jnp.maximum(accM[sl], r)
                        plsc.addupdate(accS.at[sl], r)
                    if with_counts:
                        cs = pl.ds(jnp.right_shift(p, PACK_SHIFT) << 4, 16)
                        plsc.addupdate(accC.at[cs], ones)
                return 0

            lax.fori_loop(0, BR // 16, _pgroup, 0)

        def super_body(s, _):
            pltpu.sync_copy(
                pe_hbm.at[pl.ds(pl.multiple_of(w * E + s * SUP, SUP), SUP)],
                pkS)
            unpack(SUP)
            gather_start(0, rows0, sem0)

            def bpair(bp, _):
                sbase = w * E + s * SUP
                gather_wait(rows0, sem0)
                gather_start(2 * bp + 1, rows1, sem1)
                process(rows0, (2 * bp) * BR, sbase + (2 * bp) * BR)
                gather_wait(rows1, sem1)

                @pl.when(bp < SUP // (2 * BR) - 1)
                def _():
                    gather_start(2 * bp + 2, rows0, sem0)

                process(rows1, (2 * bp + 1) * BR, sbase + (2 * bp + 1) * BR)
                return 0

            lax.fori_loop(0, SUP // (2 * BR), bpair, 0)
            return 0

        lax.fori_loop(0, nsup, super_body, 0)

        def tail_body(t, _):
            base_e = nsup * SUP + t * (2 * BR)
            pltpu.sync_copy(
                pe_hbm.at[pl.ds(pl.multiple_of(w * E + base_e, 2 * BR),
                                2 * BR)],
                pkS.at[pl.ds(0, 2 * BR)])
            unpack(2 * BR)
            gather_start(0, rows0, sem0)
            gather_wait(rows0, sem0)
            gather_start(1, rows1, sem1)
            process(rows0, 0, w * E + base_e)
            gather_wait(rows1, sem1)
            process(rows1, BR, w * E + base_e + BR)
            return 0

        lax.fori_loop(0, rem, tail_body, 0)

        woff = w * (NPT * 128)
        pltpu.sync_copy(accM.at[pl.ds(0, NPT * 128)],
                        mmax_hbm.at[pl.ds(pl.multiple_of(woff, 128),
                                          NPT * 128)])
        pltpu.sync_copy(accS.at[pl.ds(0, NPT * 128)],
                        msum_hbm.at[pl.ds(pl.multiple_of(woff, 128),
                                          NPT * 128)])
        if with_counts:
            pltpu.sync_copy(
                accC.at[pl.ds(0, NPT * 16)],
                cnt_hbm.at[pl.ds(pl.multiple_of(w * (NPT * 16), 16),
                                 NPT * 16)])

    return body


def _sc_agg(h, pe, cnts, with_counts):
    out_type = [
        jax.ShapeDtypeStruct((NOUT_PAD * 128,), jnp.float32),
        jax.ShapeDtypeStruct((NOUT_PAD * 128,), jnp.float32),
    ]
    scratch = [
        pltpu.VMEM((ACC,), jnp.float32),
        pltpu.VMEM((ACC,), jnp.float32),
    ]
    if with_counts:
        out_type.append(jax.ShapeDtypeStruct((NOUT_PAD * 16,), jnp.float32))
        scratch.append(pltpu.VMEM((NROWS * 16,), jnp.float32))
    scratch += [
        pltpu.VMEM((2048,), jnp.int32),
        pltpu.VMEM((2048,), jnp.int32),
        pltpu.VMEM((BR, 128), jnp.float32),
        pltpu.VMEM((BR, 128), jnp.float32),
        pltpu.VMEM((16,), jnp.int32),
        pltpu.SemaphoreType.DMA,
        pltpu.SemaphoreType.DMA,
    ]
    k = pl.kernel(
        _make_agg_body(with_counts),
        out_type=out_type,
        mesh=_mesh,
        scratch_types=scratch,
        compiler_params=_sc_params(),
    )
    return k(h, pe, cnts)


def _ln(y, g, b):
    mu = jnp.mean(y, axis=-1, keepdims=True)
    d = y - mu
    var = jnp.mean(d * d, axis=-1, keepdims=True)
    return g * d / jnp.sqrt(var + 1e-5) + b


def _silu(y):
    return y * jax.nn.sigmoid(y)


_HI = jax.lax.Precision.HIGHEST


def _adapter_body(x_ref, w_ref, b_ref, o_ref):
    o_ref[...] = jnp.dot(x_ref[...], w_ref[...], precision=_HI) + b_ref[...]


def _tc_adapter(x, WaT, ba):
    return pl.pallas_call(
        _adapter_body,
        grid=(N // _BLK,),
        in_specs=[
            pl.BlockSpec((_BLK, DIN), lambda i: (i, 0)),
            pl.BlockSpec((DIN, H), lambda i: (0, 0)),
            pl.BlockSpec((1, H), lambda i: (0, 0)),
        ],
        out_specs=pl.BlockSpec((_BLK, H), lambda i: (i, 0)),
        out_shape=jax.ShapeDtypeStruct((N, H), jnp.float32),
    )(x, WaT, ba.reshape(1, H))


def _dense_body(mmax_ref, msum_ref, cnt_ref, hin_ref, res_ref, wlmax_ref,
                wlmean_ref, bl_ref, wr_ref, g_ref, b_ref, o_ref):
    mmax = mmax_ref[...]
    mmaxf = jnp.where(jnp.isneginf(mmax), 0.0, mmax)
    cnt = cnt_ref[...][:, 0:1]
    mean = msum_ref[...] / jnp.maximum(cnt, 1.0)
    y = (jnp.dot(mmaxf, wlmax_ref[...], precision=_HI)
         + jnp.dot(mean, wlmean_ref[...], precision=_HI)
         + jnp.dot(hin_ref[...], wr_ref[...], precision=_HI)
         + bl_ref[...])
    y = _ln(y, g_ref[...], b_ref[...])
    o_ref[...] = _silu(y + res_ref[...])


def _tc_dense(mmax, msum, cntc, hin, res, WlTmax, WlTmean, bl, WrT, g, b):
    return pl.pallas_call(
        _dense_body,
        grid=(N // _BLK,),
        in_specs=[
            pl.BlockSpec((_BLK, H), lambda i: (i, 0)),
            pl.BlockSpec((_BLK, H), lambda i: (i, 0)),
            pl.BlockSpec((_BLK, 16), lambda i: (i, 0)),
            pl.BlockSpec((_BLK, H), lambda i: (i, 0)),
            pl.BlockSpec((_BLK, H), lambda i: (i, 0)),
            pl.BlockSpec((H, H), lambda i: (0, 0)),
            pl.BlockSpec((H, H), lambda i: (0, 0)),
            pl.BlockSpec((1, H), lambda i: (0, 0)),
            pl.BlockSpec((H, H), lambda i: (0, 0)),
            pl.BlockSpec((1, H), lambda i: (0, 0)),
            pl.BlockSpec((1, H), lambda i: (0, 0)),
        ],
        out_specs=pl.BlockSpec((_BLK, H), lambda i: (i, 0)),
        out_shape=jax.ShapeDtypeStruct((N, H), jnp.float32),
    )(mmax, msum, cntc, hin, res, WlTmax, WlTmean, bl.reshape(1, H), WrT,
      g.reshape(1, H), b.reshape(1, H))


def _mlp_body(h_ref, w1_ref, b1_ref, g_ref, b_ref, w2_ref, b2_ref, o_ref):
    z = jnp.dot(h_ref[...], w1_ref[...], precision=_HI) + b1_ref[...]
    z = _silu(z)
    z = _ln(z, g_ref[...], b_ref[...])
    o_ref[...] = jnp.dot(z, w2_ref[...], precision=_HI) + b2_ref[...]


def _tc_mlp(h, W1T, b1m, gm, bm, W2T, b2m):
    return pl.pallas_call(
        _mlp_body,
        grid=(N // _BLK,),
        in_specs=[
            pl.BlockSpec((_BLK, H), lambda i: (i, 0)),
            pl.BlockSpec((H, 4 * H), lambda i: (0, 0)),
            pl.BlockSpec((1, 4 * H), lambda i: (0, 0)),
            pl.BlockSpec((1, 4 * H), lambda i: (0, 0)),
            pl.BlockSpec((1, 4 * H), lambda i: (0, 0)),
            pl.BlockSpec((4 * H, H), lambda i: (0, 0)),
            pl.BlockSpec((1, H), lambda i: (0, 0)),
        ],
        out_specs=pl.BlockSpec((_BLK, H), lambda i: (i, 0)),
        out_shape=jax.ShapeDtypeStruct((N, H), jnp.float32),
    )(h, W1T, b1m.reshape(1, 4 * H), gm.reshape(1, 4 * H),
      bm.reshape(1, 4 * H), W2T, b2m.reshape(1, H))


_PERM = np.arange(128).reshape(4, 2, 16).transpose(0, 2, 1).reshape(128)


def kernel(x, edge_index, Wa, ba, Wl0, bl0, Wr0, g0, b0, Wl1, bl1, Wr1, g1, b1,
           Wl2, bl2, Wr2, g2, b2, W1, b1m, gm, bm, W2, b2m):
    pe, cnts = _sc_group(edge_index[0], edge_index[1])
    x_res = _tc_adapter(x, Wa.T, ba)

    mmax_f, msum_f, cnt_f = _sc_agg(x, pe, cnts, with_counts=True)
    mmax = mmax_f[:N * 128].reshape(N, 128)
    msum = msum_f[:N * 128].reshape(N, 128)
    cntc = cnt_f[:N * 16].reshape(N, 16)

    h = _tc_dense(mmax, msum, cntc, x, x_res,
                  Wl0[:, :DIN].T, Wl0[:, DIN:].T, bl0, Wr0.T, g0, b0)
    for (Wl, bl, Wr, g, b) in ((Wl1, bl1, Wr1, g1, b1),
                               (Wl2, bl2, Wr2, g2, b2)):
        mmax_f, msum_f = _sc_agg(h, pe, cnts, with_counts=False)
        mmax = mmax_f[:N * 128].reshape(N, 128)
        msum = msum_f[:N * 128].reshape(N, 128)
        h = _tc_dense(mmax, msum, cntc, h, h,
                      Wl[:, :H].T, Wl[:, H:].T, bl, Wr.T, g, b)
    return _tc_mlp(h, W1.T, b1m, gm, bm, W2.T, b2m)


# speedup vs baseline: 1.0154x; 1.0001x over previous
import dataclasses

import jax
import jax.numpy as jnp
import numpy as np
from jax import lax
from jax.experimental import pallas as pl
from jax.experimental.pallas import tpu as pltpu
from jax.experimental.pallas import tpu_sc as plsc

N = 10000
E = 320000
DIN = 128
H = 128

NTILES = 32
NPT = 313
NROWS = NPT + 1
MAGIC = 13401
MSHIFT = 22
PACK_SHIFT = 14
PAD_PACKED = NPT << PACK_SHIFT

CH = 2000
NCH = E // CH
BR = 128
BATCH_PAIR = 2 * BR
FLUSH = 2048
BUFCAP = 4096

NOUT_PAD = NTILES * NPT + 1
ACC = NROWS * 128

_BLK = 1000

def _mesh():
    return plsc.VectorSubcoreMesh(core_axis_name="c", subcore_axis_name="s")


def _sc_params():
    cp = pltpu.CompilerParams()
    if "needs_layout_passes" in pltpu.CompilerParams.__dataclass_fields__:
        cp = dataclasses.replace(cp, needs_layout_passes=False)
    return cp


def _wid():
    return lax.axis_index("s") * 2 + lax.axis_index("c")


def _group_body(src_hbm, dst_hbm, pe_hbm, cnts_hbm, srcbuf, dstbuf,
                srcbuf2, dstbuf2, buf, outv, gsem0, gsem1):
    w = _wid()
    w_vec = jnp.full((16,), 0, jnp.int32) + w
    base_vec = w_vec * NPT

    def start_load(c, sb, db, sem):
        off = c * CH
        pltpu.async_copy(src_hbm.at[pl.ds(off, CH)], sb, sem)
        pltpu.async_copy(dst_hbm.at[pl.ds(off, CH)], db, sem)

    def wait_load(sb, db, sem):
        pltpu.make_async_copy(src_hbm.at[pl.ds(0, CH)], sb, sem).wait()
        pltpu.make_async_copy(dst_hbm.at[pl.ds(0, CH)], db, sem).wait()

    def chunk_body(c, carry, sb, db):
        cursor_vec, hbm_cur = carry

        def vreg_body(j, cur):
            s16 = sb[pl.ds(j * 16, 16)]
            d16 = db[pl.ds(j * 16, 16)]
            own = jnp.right_shift(d16 * MAGIC, MSHIFT)
            m = own == w_vec
            dl = d16 - base_vec
            pk = jnp.left_shift(dl, PACK_SHIFT) | s16
            cum = plsc.cumsum(m.astype(jnp.int32))
            pos = cur + cum - 1
            plsc.store_scatter(buf, [pos], pk, mask=m)
            return cur + plsc.all_reduce_population_count(m)

        cursor_vec = lax.fori_loop(0, CH // 16, vreg_body, cursor_vec, unroll=8)
        cursor_s = jnp.max(cursor_vec, axis=0)
        do_flush = cursor_s >= FLUSH

        @pl.when(do_flush)
        def _():
            pltpu.sync_copy(buf.at[pl.ds(0, FLUSH)],
                            pe_hbm.at[pl.ds(pl.multiple_of(w * E + hbm_cur, FLUSH), FLUSH)])
            nshift = (cursor_s - FLUSH + 15) >> 4

            def shift_body(j, _):
                v = buf[pl.ds(FLUSH + j * 16, 16)]
                buf[pl.ds(j * 16, 16)] = v
                return 0

            lax.fori_loop(0, nshift, shift_body, 0)

        dec = jnp.where(do_flush, FLUSH, 0)
        return cursor_vec - dec, hbm_cur + dec

    def pair_body(i, carry):
        start_load(2 * i + 1, srcbuf2, dstbuf2, gsem1)
        wait_load(srcbuf, dstbuf, gsem0)
        carry = chunk_body(2 * i, carry, srcbuf, dstbuf)

        @pl.when(i < NCH // 2 - 1)
        def _():
            start_load(2 * i + 2, srcbuf, dstbuf, gsem0)

        wait_load(srcbuf2, dstbuf2, gsem1)
        return chunk_body(2 * i + 1, carry, srcbuf2, dstbuf2)

    start_load(0, srcbuf, dstbuf, gsem0)
    cursor_vec, hbm_cur = lax.fori_loop(
        0, NCH // 2, pair_body,
        (jnp.zeros((16,), jnp.int32), jnp.int32(0)))

    cursor_s = jnp.max(cursor_vec, axis=0)
    total = hbm_cur + cursor_s
    target = (total + BATCH_PAIR - 1) & ~(BATCH_PAIR - 1)
    cursor_end = target - hbm_cur
    astart = (cursor_s >> 4) << 4
    npadv = (cursor_end - astart + 15) >> 4
    iota16 = lax.iota(jnp.int32, 16)

    def pad_body(j, _):
        a = astart + j * 16
        v = buf[pl.ds(a, 16)]
        keep = (a + iota16) < cursor_s
        buf[pl.ds(a, 16)] = jnp.where(keep, v, jnp.full((16,), PAD_PACKED,
                                                        jnp.int32))
        return 0

    lax.fori_loop(0, npadv, pad_body, 0)

    nf = cursor_end >> 8

    def flush_body(j, _):
        pltpu.sync_copy(buf.at[pl.ds(pl.multiple_of(j * 256, 256), 256)],
                        pe_hbm.at[pl.ds(pl.multiple_of(w * E + hbm_cur + j * 256, 256), 256)])
        return 0

    lax.fori_loop(0, nf, flush_body, 0)

    outv[...] = jnp.zeros((16,), jnp.int32) + target
    pltpu.sync_copy(outv, cnts_hbm.at[pl.ds(pl.multiple_of(w * 16, 16), 16)])


def _sc_group(src, dst):
    k = pl.kernel(
        _group_body,
        out_type=[
            jax.ShapeDtypeStruct((NTILES * E,), jnp.int32),
            jax.ShapeDtypeStruct((NTILES * 16,), jnp.int32),
        ],
        mesh=_mesh(),
        scratch_types=[
            pltpu.VMEM((CH,), jnp.int32),
            pltpu.VMEM((CH,), jnp.int32),
            pltpu.VMEM((CH,), jnp.int32),
            pltpu.VMEM((CH,), jnp.int32),
            pltpu.VMEM((BUFCAP,), jnp.int32),
            pltpu.VMEM((16,), jnp.int32),
            pltpu.SemaphoreType.DMA,
            pltpu.SemaphoreType.DMA,
        ],
        compiler_params=_sc_params(),
    )
    return k(src, dst)


def _make_agg_body(with_counts):
    SUP = 2048

    def body(h_hbm, pe_hbm, cnts_hbm, *refs):
        if with_counts:
            (mmax_hbm, msum_hbm, cnt_hbm, accM, accS, accC,
             pkS, idxS, rows0, rows1, cbuf, sem0, sem1) = refs
        else:
            (mmax_hbm, msum_hbm, accM, accS,
             pkS, idxS, rows0, rows1, cbuf, sem0, sem1) = refs
        w = _wid()
        neg_inf = jnp.full((16,), -jnp.inf, jnp.float32)
        zeros = jnp.zeros((16,), jnp.float32)
        ones = jnp.ones((16,), jnp.float32)
        fmask = jnp.full((16,), (1 << PACK_SHIFT) - 1, jnp.int32)

        @pl.loop(0, ACC, step=16)
        def _(i):
            accM[pl.ds(i, 16)] = neg_inf
            accS[pl.ds(i, 16)] = zeros

        if with_counts:
            @pl.loop(0, NROWS * 16, step=16)
            def _(i):
                accC[pl.ds(i, 16)] = zeros

        pltpu.sync_copy(cnts_hbm.at[pl.ds(pl.multiple_of(w * 16, 16), 16)],
                        cbuf)
        npairs = jnp.max(cbuf[...], axis=0) >> 8
        nsup = npairs >> 3
        rem = npairs & 7

        def unpack(n):
            @pl.loop(0, n, step=16)
            def _(j):
                idxS[pl.ds(j, 16)] = pkS[pl.ds(j, 16)] & fmask

        def gather_start(b, rows, sem):
            pltpu.async_copy(h_hbm.at[idxS.at[pl.ds(b * BR, BR)]], rows, sem)

        def gather_wait(rows, sem):
            pltpu.make_async_copy(h_hbm.at[idxS.at[pl.ds(0, BR)]], rows,
                                  sem).wait()

        def process(rows, pkbase, absbase):
            def _pgroup(g, _):
                pk16 = pkS[pl.ds(pkbase + g * 16, 16)]
                for l in range(16):
                    p = pk16[l]
                    e = g * 16 + l
                    fb = jnp.right_shift(p, PACK_SHIFT) << 7
                    for kk in range(8):
                        sl = pl.ds(fb + kk * 16, 16)
                        r = rows[e, pl.ds(kk * 16, 16)]
                        accM[sl] = jnp.maximum(accM[sl], r)
                        plsc.addupdate(accS.at[sl], r)
                    if with_counts:
                        cs = pl.ds(jnp.right_shift(p, PACK_SHIFT) << 4, 16)
                        plsc.addupdate(accC.at[cs], ones)
                return 0

            lax.fori_loop(0, BR // 16, _pgroup, 0)

        def super_body(s, _):
            pltpu.sync_copy(
                pe_hbm.at[pl.ds(pl.multiple_of(w * E + s * SUP, SUP), SUP)],
                pkS)
            unpack(SUP)
            gather_start(0, rows0, sem0)

            def bpair(bp, _):
                sbase = w * E + s * SUP
                gather_wait(rows0, sem0)
                gather_start(2 * bp + 1, rows1, sem1)
                process(rows0, (2 * bp) * BR, sbase + (2 * bp) * BR)
                gather_wait(rows1, sem1)

                @pl.when(bp < SUP // (2 * BR) - 1)
                def _():
                    gather_start(2 * bp + 2, rows0, sem0)

                process(rows1, (2 * bp + 1) * BR, sbase + (2 * bp + 1) * BR)
                return 0

            lax.fori_loop(0, SUP // (2 * BR), bpair, 0)
            return 0

        lax.fori_loop(0, nsup, super_body, 0)

        def tail_body(t, _):
            base_e = nsup * SUP + t * (2 * BR)
            pltpu.sync_copy(
                pe_hbm.at[pl.ds(pl.multiple_of(w * E + base_e, 2 * BR),
                                2 * BR)],
                pkS.at[pl.ds(0, 2 * BR)])
            unpack(2 * BR)
            gather_start(0, rows0, sem0)
            gather_wait(rows0, sem0)
            gather_start(1, rows1, sem1)
            process(rows0, 0, w * E + base_e)
            gather_wait(rows1, sem1)
            process(rows1, BR, w * E + base_e + BR)
            return 0

        lax.fori_loop(0, rem, tail_body, 0)

        woff = w * (NPT * 128)
        pltpu.sync_copy(accM.at[pl.ds(0, NPT * 128)],
                        mmax_hbm.at[pl.ds(pl.multiple_of(woff, 128),
                                          NPT * 128)])
        pltpu.sync_copy(accS.at[pl.ds(0, NPT * 128)],
                        msum_hbm.at[pl.ds(pl.multiple_of(woff, 128),
                                          NPT * 128)])
        if with_counts:
            pltpu.sync_copy(
                accC.at[pl.ds(0, NPT * 16)],
                cnt_hbm.at[pl.ds(pl.multiple_of(w * (NPT * 16), 16),
                                 NPT * 16)])

    return body


def _sc_agg(h, pe, cnts, with_counts):
    out_type = [
        jax.ShapeDtypeStruct((NOUT_PAD * 128,), jnp.float32),
        jax.ShapeDtypeStruct((NOUT_PAD * 128,), jnp.float32),
    ]
    scratch = [
        pltpu.VMEM((ACC,), jnp.float32),
        pltpu.VMEM((ACC,), jnp.float32),
    ]
    if with_counts:
        out_type.append(jax.ShapeDtypeStruct((NOUT_PAD * 16,), jnp.float32))
        scratch.append(pltpu.VMEM((NROWS * 16,), jnp.float32))
    scratch += [
        pltpu.VMEM((2048,), jnp.int32),
        pltpu.VMEM((2048,), jnp.int32),
        pltpu.VMEM((BR, 128), jnp.float32),
        pltpu.VMEM((BR, 128), jnp.float32),
        pltpu.VMEM((16,), jnp.int32),
        pltpu.SemaphoreType.DMA,
        pltpu.SemaphoreType.DMA,
    ]
    k = pl.kernel(
        _make_agg_body(with_counts),
        out_type=out_type,
        mesh=_mesh(),
        scratch_types=scratch,
        compiler_params=_sc_params(),
    )
    return k(h, pe, cnts)


def _ln(y, g, b):
    mu = jnp.mean(y, axis=-1, keepdims=True)
    d = y - mu
    var = jnp.mean(d * d, axis=-1, keepdims=True)
    return g * d / jnp.sqrt(var + 1e-5) + b


def _silu(y):
    return y * jax.nn.sigmoid(y)


_HI = jax.lax.Precision.HIGHEST


def _adapter_body(x_ref, w_ref, b_ref, o_ref):
    o_ref[...] = jnp.dot(x_ref[...], w_ref[...], precision=_HI) + b_ref[...]


def _tc_adapter(x, WaT, ba):
    return pl.pallas_call(
        _adapter_body,
        grid=(N // _BLK,),
        in_specs=[
            pl.BlockSpec((_BLK, DIN), lambda i: (i, 0)),
            pl.BlockSpec((DIN, H), lambda i: (0, 0)),
            pl.BlockSpec((1, H), lambda i: (0, 0)),
        ],
        out_specs=pl.BlockSpec((_BLK, H), lambda i: (i, 0)),
        out_shape=jax.ShapeDtypeStruct((N, H), jnp.float32),
    )(x, WaT, ba.reshape(1, H))


def _dense_body(mmax_ref, msum_ref, cnt_ref, hin_ref, res_ref, wlmax_ref,
                wlmean_ref, bl_ref, wr_ref, g_ref, b_ref, o_ref):
    mmax = mmax_ref[...]
    mmaxf = jnp.where(jnp.isneginf(mmax), 0.0, mmax)
    cnt = cnt_ref[...][:, 0:1]
    mean = msum_ref[...] / jnp.maximum(cnt, 1.0)
    y = (jnp.dot(mmaxf, wlmax_ref[...], precision=_HI)
         + jnp.dot(mean, wlmean_ref[...], precision=_HI)
         + jnp.dot(hin_ref[...], wr_ref[...], precision=_HI)
         + bl_ref[...])
    y = _ln(y, g_ref[...], b_ref[...])
    o_ref[...] = _silu(y + res_ref[...])


def _tc_dense(mmax, msum, cntc, hin, res, WlTmax, WlTmean, bl, WrT, g, b):
    return pl.pallas_call(
        _dense_body,
        grid=(N // _BLK,),
        in_specs=[
            pl.BlockSpec((_BLK, H), lambda i: (i, 0)),
            pl.BlockSpec((_BLK, H), lambda i: (i, 0)),
            pl.BlockSpec((_BLK, 16), lambda i: (i, 0)),
            pl.BlockSpec((_BLK, H), lambda i: (i, 0)),
            pl.BlockSpec((_BLK, H), lambda i: (i, 0)),
            pl.BlockSpec((H, H), lambda i: (0, 0)),
            pl.BlockSpec((H, H), lambda i: (0, 0)),
            pl.BlockSpec((1, H), lambda i: (0, 0)),
            pl.BlockSpec((H, H), lambda i: (0, 0)),
            pl.BlockSpec((1, H), lambda i: (0, 0)),
            pl.BlockSpec((1, H), lambda i: (0, 0)),
        ],
        out_specs=pl.BlockSpec((_BLK, H), lambda i: (i, 0)),
        out_shape=jax.ShapeDtypeStruct((N, H), jnp.float32),
    )(mmax, msum, cntc, hin, res, WlTmax, WlTmean, bl.reshape(1, H), WrT,
      g.reshape(1, H), b.reshape(1, H))


def _mlp_body(h_ref, w1_ref, b1_ref, g_ref, b_ref, w2_ref, b2_ref, o_ref):
    z = jnp.dot(h_ref[...], w1_ref[...], precision=_HI) + b1_ref[...]
    z = _silu(z)
    z = _ln(z, g_ref[...], b_ref[...])
    o_ref[...] = jnp.dot(z, w2_ref[...], precision=_HI) + b2_ref[...]


def _tc_mlp(h, W1T, b1m, gm, bm, W2T, b2m):
    return pl.pallas_call(
        _mlp_body,
        grid=(N // _BLK,),
        in_specs=[
            pl.BlockSpec((_BLK, H), lambda i: (i, 0)),
            pl.BlockSpec((H, 4 * H), lambda i: (0, 0)),
            pl.BlockSpec((1, 4 * H), lambda i: (0, 0)),
            pl.BlockSpec((1, 4 * H), lambda i: (0, 0)),
            pl.BlockSpec((1, 4 * H), lambda i: (0, 0)),
            pl.BlockSpec((4 * H, H), lambda i: (0, 0)),
            pl.BlockSpec((1, H), lambda i: (0, 0)),
        ],
        out_specs=pl.BlockSpec((_BLK, H), lambda i: (i, 0)),
        out_shape=jax.ShapeDtypeStruct((N, H), jnp.float32),
    )(h, W1T, b1m.reshape(1, 4 * H), gm.reshape(1, 4 * H),
      bm.reshape(1, 4 * H), W2T, b2m.reshape(1, H))


_PERM = np.arange(128).reshape(4, 2, 16).transpose(0, 2, 1).reshape(128)


def kernel(x, edge_index, Wa, ba, Wl0, bl0, Wr0, g0, b0, Wl1, bl1, Wr1, g1, b1,
           Wl2, bl2, Wr2, g2, b2, W1, b1m, gm, bm, W2, b2m):
    pe, cnts = _sc_group(edge_index[0], edge_index[1])
    x_res = _tc_adapter(x, Wa.T, ba)

    mmax_f, msum_f, cnt_f = _sc_agg(x, pe, cnts, with_counts=True)
    mmax = mmax_f[:N * 128].reshape(N, 128)
    msum = msum_f[:N * 128].reshape(N, 128)
    cntc = cnt_f[:N * 16].reshape(N, 16)

    h = _tc_dense(mmax, msum, cntc, x, x_res,
                  Wl0[:, :DIN].T, Wl0[:, DIN:].T, bl0, Wr0.T, g0, b0)
    for (Wl, bl, Wr, g, b) in ((Wl1, bl1, Wr1, g1, b1),
                               (Wl2, bl2, Wr2, g2, b2)):
        mmax_f, msum_f = _sc_agg(h, pe, cnts, with_counts=False)
        mmax = mmax_f[:N * 128].reshape(N, 128)
        msum = msum_f[:N * 128].reshape(N, 128)
        h = _tc_dense(mmax, msum, cntc, h, h,
                      Wl[:, :H].T, Wl[:, H:].T, bl, Wr.T, g, b)
    return _tc_mlp(h, W1.T, b1m, gm, bm, W2.T, b2m)
